# baseline (speedup 1.0000x reference)
import jax
import jax.numpy as jnp
from jax.experimental import pallas as pl
from jax.experimental.pallas import tpu as pltpu

_N_PTS0 = 32
_N_PTS1 = 8
_SF = _N_PTS0 // _N_PTS1


def _fc1_body(f_ref, w_ref, b_ref, o_ref):
    a = f_ref[0]
    o = jax.lax.dot_general(a, w_ref[...], (((0,), (1,)), ((), ())),
                            preferred_element_type=jnp.float32,
                            precision=jax.lax.Precision.HIGHEST)
    o_ref[0] = o + b_ref[...]


def _gather_body(idx_ref, w4_ref, x_hbm, o_ref, xv, sem, *, lines, w_img):
    t = pl.program_id(1)

    @pl.when(t == 0)
    def _copy_in():
        b = pl.program_id(0)
        cp = pltpu.make_async_copy(x_hbm.at[b], xv, sem)
        cp.start()
        cp.wait()

    def line_body(l, carry):
        base = l * _N_PTS0
        vals = []
        for i in range(_N_PTS0):
            k = base + i
            n = idx_ref[0, 0, k]
            wa = w4_ref[0, 0, k]
            wb = w4_ref[0, 1, k]
            wc = w4_ref[0, 2, k]
            wd = w4_ref[0, 3, k]
            v = (xv[n] * wa + xv[n + w_img] * wb
                 + xv[n + 1] * wc + xv[n + w_img + 1] * wd)
            vals.append(v)
        chunks = [jnp.maximum(jnp.maximum(vals[_SF * j], vals[_SF * j + 1]),
                              jnp.maximum(vals[_SF * j + 2], vals[_SF * j + 3]))
                  for j in range(_N_PTS1)]
        o_ref[0, l] = jnp.concatenate(chunks, axis=-1)
        return carry

    jax.lax.fori_loop(0, lines, line_body, 0)


def _mlp_body(f_ref, w1_ref, b1_ref, w2_ref, b2_ref, w3_ref, b3_ref, o_ref):
    dn = (((1,), (1,)), ((), ()))
    h = jax.lax.dot_general(f_ref[...], w1_ref[...], dn,
                            preferred_element_type=jnp.float32,
                            precision=jax.lax.Precision.HIGHEST)
    h = jnp.maximum(h + b1_ref[...], 0.0)
    h = jax.lax.dot_general(h, w2_ref[...], dn,
                            preferred_element_type=jnp.float32,
                            precision=jax.lax.Precision.HIGHEST)
    h = jnp.maximum(h + b2_ref[...], 0.0)
    o_ref[...] = jax.lax.dot_general(h, w3_ref[...], dn,
                                     preferred_element_type=jnp.float32,
                                     precision=jax.lax.Precision.HIGHEST) + b3_ref[...]


def kernel(feature, p, w_fc1, b_fc1, w1, b1, w2, b2, w3, b3):
    B, C, H, W = feature.shape
    L = p.shape[1]
    D = w_fc1.shape[0]
    HW = H * W
    DIM_FC = w1.shape[0]
    N_OUT = w3.shape[0]

    hwt = 2048 if HW % 2048 == 0 else HW
    fmap = feature.reshape(B, C, HW)
    x = pl.pallas_call(
        _fc1_body,
        grid=(B, HW // hwt),
        in_specs=[
            pl.BlockSpec((1, C, hwt), lambda b, t: (b, 0, t)),
            pl.BlockSpec((D, C), lambda b, t: (0, 0)),
            pl.BlockSpec((1, D), lambda b, t: (0, 0)),
        ],
        out_specs=pl.BlockSpec((1, hwt, D), lambda b, t: (b, t, 0)),
        out_shape=jax.ShapeDtypeStruct((B, HW, D), jnp.float32),
        compiler_params=pltpu.CompilerParams(
            dimension_semantics=("parallel", "arbitrary")),
    )(fmap, w_fc1, b_fc1.reshape(1, D))

    lam = jnp.linspace(0.0, 1.0, _N_PTS0, dtype=jnp.float32)[:, None]
    pts = p[:, :, 0:1, :] * lam + p[:, :, 1:2, :] * (1.0 - lam) - 0.5
    pts = pts.reshape(B, L * _N_PTS0, 2)
    px, py = pts[..., 0], pts[..., 1]
    fH, fW = float(H - 1), float(W - 1)
    px0 = jnp.clip(jnp.floor(px), 0.0, fH)
    py0 = jnp.clip(jnp.floor(py), 0.0, fW)
    px1 = jnp.clip(px0 + 1.0, 0.0, fH)
    py1 = jnp.clip(py0 + 1.0, 0.0, fW)
    i0 = px0.astype(jnp.int32)
    j0 = py0.astype(jnp.int32)
    idx = i0 * W + j0
    w4 = jnp.stack([(px1 - px) * (py1 - py),
                    (px - px0) * (py1 - py),
                    (px1 - px) * (py - py0),
                    (px - px0) * (py - py0)], axis=1)

    lt = next(c for c in (125, 25, 5, 1) if L % c == 0)
    pts_t = lt * _N_PTS0
    n_t = L // lt
    idx_t = idx.reshape(B * n_t, 1, pts_t)
    w4_t = (w4.reshape(B, 4, n_t, pts_t).transpose(0, 2, 1, 3)
            .reshape(B * n_t, 4, pts_t))
    import functools
    gbody = functools.partial(_gather_body, lines=lt, w_img=W)
    feat = pl.pallas_call(
        gbody,
        grid=(B, n_t),
        in_specs=[
            pl.BlockSpec((1, 1, pts_t), lambda b, t: (b * n_t + t, 0, 0),
                         memory_space=pltpu.SMEM),
            pl.BlockSpec((1, 4, pts_t), lambda b, t: (b * n_t + t, 0, 0),
                         memory_space=pltpu.SMEM),
            pl.BlockSpec(memory_space=pl.ANY),
        ],
        out_specs=pl.BlockSpec((1, lt, 1, _N_PTS1 * D),
                               lambda b, t: (b, t, 0, 0)),
        out_shape=jax.ShapeDtypeStruct((B, L, 1, _N_PTS1 * D), jnp.float32),
        scratch_shapes=[pltpu.VMEM((HW, 1, D), jnp.float32),
                        pltpu.SemaphoreType.DMA],
        compiler_params=pltpu.CompilerParams(
            dimension_semantics=("parallel", "arbitrary"),
            vmem_limit_bytes=50 * 1024 * 1024),
    )(idx_t, w4_t, x.reshape(B, HW, 1, D))
    feat = feat.reshape(B * L, _N_PTS1 * D)

    w1p = w1.reshape(DIM_FC, D, _N_PTS1).transpose(0, 2, 1).reshape(DIM_FC, _N_PTS1 * D)
    w3p = jnp.zeros((128, DIM_FC), jnp.float32).at[:N_OUT].set(w3)
    b3p = jnp.zeros((1, 128), jnp.float32).at[0, :N_OUT].set(b3)
    rows = B * L
    mt = next(c for c in (400, 40, 8, 1) if rows % c == 0)
    logits = pl.pallas_call(
        _mlp_body,
        grid=(rows // mt,),
        in_specs=[
            pl.BlockSpec((mt, _N_PTS1 * D), lambda i: (i, 0)),
            pl.BlockSpec((DIM_FC, _N_PTS1 * D), lambda i: (0, 0)),
            pl.BlockSpec((1, DIM_FC), lambda i: (0, 0)),
            pl.BlockSpec((DIM_FC, DIM_FC), lambda i: (0, 0)),
            pl.BlockSpec((1, DIM_FC), lambda i: (0, 0)),
            pl.BlockSpec((128, DIM_FC), lambda i: (0, 0)),
            pl.BlockSpec((1, 128), lambda i: (0, 0)),
        ],
        out_specs=pl.BlockSpec((mt, 128), lambda i: (i, 0)),
        out_shape=jax.ShapeDtypeStruct((rows, 128), jnp.float32),
        compiler_params=pltpu.CompilerParams(
            dimension_semantics=("parallel",)),
    )(feat, w1p, b1.reshape(1, DIM_FC), w2, b2.reshape(1, DIM_FC), w3p, b3p)
    return logits[:, :N_OUT]
